# baseline (speedup 1.0000x reference)
import functools

import jax
import jax.numpy as jnp
import numpy as np
from jax import lax
from jax.experimental import pallas as pl
from jax.experimental.pallas import tpu as pltpu
from jax.experimental.pallas import tpu_sc as plsc

EPS = 1e-06
PRED_LENGTH = 12
K = 20
T = 12
L = 2 * T
B = 10000
BP = 10240
BBA = 512

_PERM = np.zeros((2 * BBA, 2 * BBA), np.float32)
for _j in range(2 * BBA):
    _PERM[(_j % 2) * BBA + _j // 2, _j] = 1.0


def _main_body(mu_ref, sg_ref, l1_ref, sc_ref, y_ref, pi_ref, po_ref, p_ref,
               s1_ref, sf1_ref, s2_ref, sf2_ref, sums_ref):
    j = pl.program_id(0)
    bb = y_ref.shape[0]
    yt = lax.transpose(y_ref[...], (1, 0))
    kio = lax.broadcasted_iota(jnp.int32, (K, bb), 0)
    bglob = j * bb + lax.broadcasted_iota(jnp.int32, (1, bb), 1)
    valid = bglob < B
    validc = lax.transpose(valid, (1, 0))
    even = (lax.broadcasted_iota(jnp.int32, (L, bb), 0) % 2 == 0
            ).astype(jnp.float32)

    def head(ref, sgref):
        tks, l2s, fs = [], [], []
        for k in range(K):
            tk = lax.transpose(ref[k], (1, 0))
            diff = tk - yt
            e = diff * diff
            s = e + pltpu.roll(e, shift=L - 1, axis=0)
            d = jnp.sqrt(s) * even
            tks.append(tk)
            l2s.append(jnp.sum(d, axis=0, keepdims=True))
            fs.append(jnp.sqrt(s[L - 2:L - 1, :]))
        l2 = jnp.concatenate(l2s, axis=0)
        f = jnp.concatenate(fs, axis=0)
        mn = jnp.min(l2, axis=0, keepdims=True)
        am = jnp.min(jnp.where(l2 == mn, kio, K), axis=0, keepdims=True)
        mnf = jnp.min(f, axis=0, keepdims=True)
        amf = jnp.min(jnp.where(f == mnf, kio, K), axis=0, keepdims=True)

        mu_ade = jnp.zeros_like(tks[0])
        mu_fde = jnp.zeros_like(tks[0])
        for k in range(K):
            mu_ade = mu_ade + jnp.where(am == k, tks[k], 0.0)
            mu_fde = mu_fde + jnp.where(amf == k, tks[k], 0.0)

        am_col = lax.transpose(am, (1, 0))
        sg_ade = jnp.zeros((bb, L), jnp.float32)
        for k in range(K):
            sg_ade = sg_ade + jnp.where(am_col == k, sgref[k], 0.0)
        sg_t = lax.transpose(sg_ade, (1, 0))

        sc = jnp.maximum(sg_t, EPS)
        nll = jnp.log(2.0 * sc) + jnp.abs(yt - mu_ade) / sc
        reg = jnp.sum(jnp.where(valid, nll, 0.0))
        return l2, mu_ade, mu_fde, reg

    l2_1, a1, f1, reg1 = head(mu_ref, sg_ref)
    _, a2, f2, reg2 = head(l1_ref, sc_ref)

    z = -l2_1 / PRED_LENGTH
    z = z - jnp.max(z, axis=0, keepdims=True)
    ez = jnp.exp(z)
    st = ez / jnp.sum(ez, axis=0, keepdims=True)
    pi = lax.transpose(pi_ref[...], (1, 0))
    pim = pi - jnp.max(pi, axis=0, keepdims=True)
    lsm = pim - jnp.log(jnp.sum(jnp.exp(pim), axis=0, keepdims=True))
    cls_b = jnp.sum(st * lsm, axis=0, keepdims=True)
    tr_sum = jnp.sum(jnp.where(valid, cls_b, 0.0))

    po = po_ref[...]
    perm = p_ref[...]

    def build(at, o_ref):
        xs = [jnp.concatenate([at[2 * t:2 * t + 1, :],
                               at[2 * t + 1:2 * t + 2, :]], axis=1)
              for t in range(T)]
        x = jnp.concatenate(xs, axis=0)
        v = lax.dot(x, perm, precision=lax.Precision.HIGHEST,
                    preferred_element_type=jnp.float32)
        o_ref[...] = jnp.concatenate([po, v], axis=0)

    build(a1, s1_ref)
    build(f1, sf1_ref)
    build(a2, s2_ref)
    build(f2, sf2_ref)

    lane1 = lax.broadcasted_iota(jnp.int32, (1, 128), 1)

    @pl.when(j == 0)
    def _():
        sums_ref[...] = jnp.zeros_like(sums_ref)

    sums_ref[...] += (jnp.where(lane1 == 0, tr_sum, 0.0)
                      + jnp.where(lane1 == 1, reg1, 0.0)
                      + jnp.where(lane1 == 2, reg2, 0.0))


@jax.jit
def kernel(out_mu, out_sigma, out_pi, loc1, scale1, y, pre_obs):
    mu3 = out_mu.reshape(K, B, L)
    sg3 = out_sigma.reshape(K, B, L)
    l13 = loc1.reshape(K, B, L)
    sc3 = scale1.reshape(K, B, L)
    y2 = y.reshape(B, L)

    grid = (BP // BBA,)
    big = pl.BlockSpec((K, BBA, L), lambda j: (0, j, 0))
    yrow = pl.BlockSpec((BBA, L), lambda j: (j, 0))
    pirow = pl.BlockSpec((BBA, K), lambda j: (j, 0))
    obs = pre_obs.shape[0]
    porow = pl.BlockSpec((obs, 2 * BBA), lambda j: (0, j))
    pcst = pl.BlockSpec((2 * BBA, 2 * BBA), lambda j: (0, 0))
    orow = pl.BlockSpec((obs + T, 2 * BBA), lambda j: (0, j))
    acc = pl.BlockSpec((1, 128), lambda j: (0, 0))
    oshape = jax.ShapeDtypeStruct((obs + T, 2 * B), jnp.float32)

    t1, t2, t3, t4, sums = pl.pallas_call(
        _main_body,
        grid=grid,
        in_specs=[big, big, big, big, yrow, pirow, porow, pcst],
        out_specs=[orow, orow, orow, orow, acc],
        out_shape=[
            oshape, oshape, oshape, oshape,
            jax.ShapeDtypeStruct((1, 128), jnp.float32),
        ],
    )(mu3, sg3, l13, sc3, y2, out_pi, pre_obs.reshape(obs, 2 * B),
      jnp.asarray(_PERM))

    loss0 = sums[0, 1] / (B * L) - sums[0, 0] / B
    loss1 = sums[0, 2] / (B * L)

    def tra(t):
        return t.reshape(obs + T, B, 2)

    return (loss0, loss1, tra(t1), tra(t2), tra(t3), tra(t4))


# speedup vs baseline: 1.1432x; 1.1432x over previous
import jax
import jax.numpy as jnp
from jax import lax
from jax.experimental import pallas as pl
from jax.experimental.pallas import tpu as pltpu

EPS = 1e-06
PRED_LENGTH = 12
K = 20
T = 12
L = 2 * T
B = 10000
BP = 10240
BBA = 256


def _main_body(mu_ref, sg_ref, l1_ref, sc_ref, y_ref, pi_ref,
               s1_ref, sf1_ref, s2_ref, sf2_ref, sums_ref):
    j = pl.program_id(0)
    bb = y_ref.shape[0]
    yt = lax.transpose(y_ref[...], (1, 0))
    kio = lax.broadcasted_iota(jnp.int32, (K, bb), 0)
    bglob = j * bb + lax.broadcasted_iota(jnp.int32, (1, bb), 1)
    valid = bglob < B
    validc = lax.transpose(valid, (1, 0))
    even = (lax.broadcasted_iota(jnp.int32, (L, bb), 0) % 2 == 0
            ).astype(jnp.float32)

    def head(ref, sgref):
        tks, l2s, fs = [], [], []
        for k in range(K):
            tk = lax.transpose(ref[k], (1, 0))
            diff = tk - yt
            e = diff * diff
            s = e + pltpu.roll(e, shift=L - 1, axis=0)
            d = jnp.sqrt(s) * even
            tks.append(tk)
            l2s.append(jnp.sum(d, axis=0, keepdims=True))
            fs.append(jnp.sqrt(s[L - 2:L - 1, :]))
        l2 = jnp.concatenate(l2s, axis=0)
        f = jnp.concatenate(fs, axis=0)
        mn = jnp.min(l2, axis=0, keepdims=True)
        am = jnp.min(jnp.where(l2 == mn, kio, K), axis=0, keepdims=True)
        mnf = jnp.min(f, axis=0, keepdims=True)
        amf = jnp.min(jnp.where(f == mnf, kio, K), axis=0, keepdims=True)

        mu_ade = jnp.zeros_like(tks[0])
        mu_fde = jnp.zeros_like(tks[0])
        for k in range(K):
            mu_ade = mu_ade + jnp.where(am == k, tks[k], 0.0)
            mu_fde = mu_fde + jnp.where(amf == k, tks[k], 0.0)

        am_col = lax.transpose(am, (1, 0))
        sg_ade = jnp.zeros((bb, L), jnp.float32)
        for k in range(K):
            sg_ade = sg_ade + jnp.where(am_col == k, sgref[k], 0.0)
        sg_t = lax.transpose(sg_ade, (1, 0))

        sc = jnp.maximum(sg_t, EPS)
        nll = jnp.log(2.0 * sc) + jnp.abs(yt - mu_ade) / sc
        reg = jnp.sum(jnp.where(valid, nll, 0.0))
        return l2, mu_ade, mu_fde, reg

    l2_1, a1, f1, reg1 = head(mu_ref, sg_ref)
    _, a2, f2, reg2 = head(l1_ref, sc_ref)

    z = -l2_1 / PRED_LENGTH
    z = z - jnp.max(z, axis=0, keepdims=True)
    ez = jnp.exp(z)
    st = ez / jnp.sum(ez, axis=0, keepdims=True)
    pi = lax.transpose(pi_ref[...], (1, 0))
    pim = pi - jnp.max(pi, axis=0, keepdims=True)
    lsm = pim - jnp.log(jnp.sum(jnp.exp(pim), axis=0, keepdims=True))
    cls_b = jnp.sum(st * lsm, axis=0, keepdims=True)
    tr_sum = jnp.sum(jnp.where(valid, cls_b, 0.0))

    s1_ref[...] = lax.transpose(a1, (1, 0))
    sf1_ref[...] = lax.transpose(f1, (1, 0))
    s2_ref[...] = lax.transpose(a2, (1, 0))
    sf2_ref[...] = lax.transpose(f2, (1, 0))

    lane1 = lax.broadcasted_iota(jnp.int32, (1, 128), 1)

    @pl.when(j == 0)
    def _():
        sums_ref[...] = jnp.zeros_like(sums_ref)

    sums_ref[...] += (jnp.where(lane1 == 0, tr_sum, 0.0)
                      + jnp.where(lane1 == 1, reg1, 0.0)
                      + jnp.where(lane1 == 2, reg2, 0.0))


@jax.jit
def kernel(out_mu, out_sigma, out_pi, loc1, scale1, y, pre_obs):
    mu3 = out_mu.reshape(K, B, L)
    sg3 = out_sigma.reshape(K, B, L)
    l13 = loc1.reshape(K, B, L)
    sc3 = scale1.reshape(K, B, L)
    y2 = y.reshape(B, L)

    grid = (BP // BBA,)
    big = pl.BlockSpec((K, BBA, L), lambda j: (0, j, 0))
    yrow = pl.BlockSpec((BBA, L), lambda j: (j, 0))
    pirow = pl.BlockSpec((BBA, K), lambda j: (j, 0))
    srow = pl.BlockSpec((BBA, L), lambda j: (j, 0))
    acc = pl.BlockSpec((1, 128), lambda j: (0, 0))
    sshape = jax.ShapeDtypeStruct((BP, L), jnp.float32)

    s1, sf1, s2, sf2, sums = pl.pallas_call(
        _main_body,
        grid=grid,
        in_specs=[big, big, big, big, yrow, pirow],
        out_specs=[srow, srow, srow, srow, acc],
        out_shape=[
            sshape, sshape, sshape, sshape,
            jax.ShapeDtypeStruct((1, 128), jnp.float32),
        ],
    )(mu3, sg3, l13, sc3, y2, out_pi)

    loss0 = sums[0, 1] / (B * L) - sums[0, 0] / B
    loss1 = sums[0, 2] / (B * L)

    def tra(s):
        samp = jnp.transpose(s[:B].reshape(B, T, 2), (1, 0, 2))
        return jnp.concatenate([pre_obs, samp], axis=0)

    return (loss0, loss1, tra(s1), tra(sf1), tra(s2), tra(sf2))


# speedup vs baseline: 1.2153x; 1.0631x over previous
import jax
import jax.numpy as jnp
from jax import lax
from jax.experimental import pallas as pl
from jax.experimental.pallas import tpu as pltpu

EPS = 1e-06
PRED_LENGTH = 12
K = 20
T = 12
L = 2 * T
B = 10000
BP = 10240
BBA = 256


def _main_body(mu_ref, sg_ref, l1_ref, sc_ref, y_ref, pi_ref,
               s1_ref, sf1_ref, s2_ref, sf2_ref, sums_ref):
    j = pl.program_id(0)
    bb = y_ref.shape[0]
    yt = lax.transpose(y_ref[...], (1, 0))
    kio = lax.broadcasted_iota(jnp.int32, (K, bb), 0)
    bglob = j * bb + lax.broadcasted_iota(jnp.int32, (1, bb), 1)
    valid = bglob < B
    validc = lax.transpose(valid, (1, 0))
    even = (lax.broadcasted_iota(jnp.int32, (L, bb), 0) % 2 == 0
            ).astype(jnp.float32)

    def head(ref, sgref):
        tks, l2s, fs = [], [], []
        for k in range(K):
            tk = lax.transpose(ref[k], (1, 0))
            diff = tk - yt
            e = diff * diff
            s = e + pltpu.roll(e, shift=L - 1, axis=0)
            d = jnp.sqrt(s) * even
            tks.append(tk)
            l2s.append(jnp.sum(d, axis=0, keepdims=True))
            fs.append(jnp.sqrt(s[L - 2:L - 1, :]))
        l2 = jnp.concatenate(l2s, axis=0)
        f = jnp.concatenate(fs, axis=0)
        mn = jnp.min(l2, axis=0, keepdims=True)
        am = jnp.min(jnp.where(l2 == mn, kio, K), axis=0, keepdims=True)
        mnf = jnp.min(f, axis=0, keepdims=True)
        amf = jnp.min(jnp.where(f == mnf, kio, K), axis=0, keepdims=True)

        mu_ade = jnp.zeros_like(tks[0])
        mu_fde = jnp.zeros_like(tks[0])
        for k in range(K):
            mu_ade = mu_ade + jnp.where(am == k, tks[k], 0.0)
            mu_fde = mu_fde + jnp.where(amf == k, tks[k], 0.0)

        am_col = lax.transpose(am, (1, 0))
        sg_ade = jnp.zeros((bb, L), jnp.float32)
        for k in range(K):
            sg_ade = sg_ade + jnp.where(am_col == k, sgref[k], 0.0)
        sg_t = lax.transpose(sg_ade, (1, 0))

        sc = jnp.maximum(sg_t, EPS)
        nll = jnp.log(2.0 * sc) + jnp.abs(yt - mu_ade) / sc
        reg = jnp.sum(jnp.where(valid, nll, 0.0))
        return l2, mu_ade, mu_fde, reg

    l2_1, a1, f1, reg1 = head(mu_ref, sg_ref)
    _, a2, f2, reg2 = head(l1_ref, sc_ref)

    z = -l2_1 / PRED_LENGTH
    z = z - jnp.max(z, axis=0, keepdims=True)
    ez = jnp.exp(z)
    st = ez / jnp.sum(ez, axis=0, keepdims=True)
    pi = lax.transpose(pi_ref[...], (1, 0))
    pim = pi - jnp.max(pi, axis=0, keepdims=True)
    lsm = pim - jnp.log(jnp.sum(jnp.exp(pim), axis=0, keepdims=True))
    cls_b = jnp.sum(st * lsm, axis=0, keepdims=True)
    tr_sum = jnp.sum(jnp.where(valid, cls_b, 0.0))

    s1_ref[...] = a1
    sf1_ref[...] = f1
    s2_ref[...] = a2
    sf2_ref[...] = f2

    lane1 = lax.broadcasted_iota(jnp.int32, (1, 128), 1)

    @pl.when(j == 0)
    def _():
        sums_ref[...] = jnp.zeros_like(sums_ref)

    sums_ref[...] += (jnp.where(lane1 == 0, tr_sum, 0.0)
                      + jnp.where(lane1 == 1, reg1, 0.0)
                      + jnp.where(lane1 == 2, reg2, 0.0))


@jax.jit
def kernel(out_mu, out_sigma, out_pi, loc1, scale1, y, pre_obs):
    mu3 = out_mu.reshape(K, B, L)
    sg3 = out_sigma.reshape(K, B, L)
    l13 = loc1.reshape(K, B, L)
    sc3 = scale1.reshape(K, B, L)
    y2 = y.reshape(B, L)

    grid = (BP // BBA,)
    big = pl.BlockSpec((K, BBA, L), lambda j: (0, j, 0))
    yrow = pl.BlockSpec((BBA, L), lambda j: (j, 0))
    pirow = pl.BlockSpec((BBA, K), lambda j: (j, 0))
    srow = pl.BlockSpec((L, BBA), lambda j: (0, j))
    acc = pl.BlockSpec((1, 128), lambda j: (0, 0))
    sshape = jax.ShapeDtypeStruct((L, BP), jnp.float32)

    s1, sf1, s2, sf2, sums = pl.pallas_call(
        _main_body,
        grid=grid,
        in_specs=[big, big, big, big, yrow, pirow],
        out_specs=[srow, srow, srow, srow, acc],
        out_shape=[
            sshape, sshape, sshape, sshape,
            jax.ShapeDtypeStruct((1, 128), jnp.float32),
        ],
    )(mu3, sg3, l13, sc3, y2, out_pi)

    loss0 = sums[0, 1] / (B * L) - sums[0, 0] / B
    loss1 = sums[0, 2] / (B * L)

    def tra(s):
        samp = jnp.transpose(s[:, :B].reshape(T, 2, B), (0, 2, 1))
        return jnp.concatenate([pre_obs, samp], axis=0)

    return (loss0, loss1, tra(s1), tra(sf1), tra(s2), tra(sf2))
